# baseline (speedup 1.0000x reference)
import functools

import jax
import jax.numpy as jnp
from jax import lax
from jax.experimental import pallas as pl
from jax.experimental.pallas import tpu as pltpu
from jax.experimental.pallas import tpu_sc as plsc

NG = 64
NN = 10000
NE = 320000
NT = 30
D = 128
IMD = 4096
PF = 147

NNP = 10240
NC = 2
NS = 16
NW = NC * NS
CH = 64
CPT = 160
EPAD = NW * CPT * CH
RPT = NNP // NS
HIGH = lax.Precision.HIGHEST

_mesh = plsc.VectorSubcoreMesh(core_axis_name="c", subcore_axis_name="s")


@functools.partial(
    pl.kernel,
    out_type=jax.ShapeDtypeStruct((NC, NNP, D), jnp.float32),
    mesh=_mesh,
    scratch_types=[
        pltpu.VMEM((CH,), jnp.int32),
        pltpu.VMEM((CH, 16), jnp.float32),
        pltpu.VMEM((CH, D), jnp.float32),
        pltpu.VMEM_SHARED((NNP, D), jnp.float32),
    ],
)
def _sc_deg(dst_hbm, wrep_hbm, zero_hbm, out_hbm,
            dst_v, wrow_v, rows_v, acc_sh):
    cid = lax.axis_index("c")
    sid = lax.axis_index("s")
    wid = sid * NC + cid
    r0 = sid * RPT
    pltpu.sync_copy(zero_hbm.at[pl.ds(0, CH)], rows_v)
    for bb in range(RPT // CH):
        pltpu.sync_copy(rows_v, acc_sh.at[pl.ds(r0 + bb * CH, CH)])
    plsc.subcore_barrier()

    def chunk_body(k, carry):
        e0 = (wid * CPT + k) * CH
        pltpu.sync_copy(dst_hbm.at[pl.ds(e0, CH)], dst_v)
        pltpu.sync_copy(wrep_hbm.at[pl.ds(e0, CH)], wrow_v)

        def edge_body(i, c2):
            wspl = wrow_v[i, pl.ds(0, 16)]
            for j in range(D // 16):
                rows_v[i, pl.ds(j * 16, 16)] = wspl
            return c2

        lax.fori_loop(0, CH, edge_body, 0)
        pltpu.sync_copy(rows_v, acc_sh.at[dst_v], add=True)
        return carry

    lax.fori_loop(0, CPT, chunk_body, 0)
    plsc.subcore_barrier()
    for bb in range(RPT // CH):
        pltpu.sync_copy(acc_sh.at[pl.ds(r0 + bb * CH, CH)], rows_v)
        pltpu.sync_copy(rows_v, out_hbm.at[cid, pl.ds(r0 + bb * CH, CH)])


@functools.partial(
    pl.kernel,
    out_type=jax.ShapeDtypeStruct((NC, NNP, D), jnp.float32),
    mesh=_mesh,
    scratch_types=[
        pltpu.VMEM((CH,), jnp.int32),
        pltpu.VMEM((CH,), jnp.int32),
        pltpu.VMEM((CH,), jnp.int32),
        pltpu.VMEM((CH,), jnp.int32),
        pltpu.VMEM((CH, 16), jnp.float32),
        pltpu.VMEM((CH, 16), jnp.float32),
        pltpu.VMEM((CH, D), jnp.float32),
        pltpu.VMEM((CH, D), jnp.float32),
        pltpu.VMEM_SHARED((NNP, D), jnp.float32),
        pltpu.SemaphoreType.DMA,
        pltpu.SemaphoreType.DMA,
    ],
)
def _sc_agg(hp_hbm, src_hbm, dst_hbm, wrep_hbm, zero_hbm, out_hbm,
            src0, src1, dst0, dst1, wrow0, wrow1, rows0, rows1,
            acc_sh, sem0, sem1):
    cid = lax.axis_index("c")
    sid = lax.axis_index("s")
    wid = sid * NC + cid
    r0 = sid * RPT
    srcs = (src0, src1)
    dsts = (dst0, dst1)
    wrows = (wrow0, wrow1)
    rows = (rows0, rows1)
    sems = (sem0, sem1)

    pltpu.sync_copy(zero_hbm.at[pl.ds(0, CH)], rows0)
    for bb in range(RPT // CH):
        pltpu.sync_copy(rows0, acc_sh.at[pl.ds(r0 + bb * CH, CH)])
    plsc.subcore_barrier()

    def fetch(k, b):
        e0 = (wid * CPT + k) * CH
        pltpu.sync_copy(src_hbm.at[pl.ds(e0, CH)], srcs[b])
        pltpu.sync_copy(wrep_hbm.at[pl.ds(e0, CH)], wrows[b])
        pltpu.sync_copy(dst_hbm.at[pl.ds(e0, CH)], dsts[b])
        pltpu.async_copy(hp_hbm.at[srcs[b]], rows[b], sems[b])

    def consume(b):
        pltpu.make_async_copy(hp_hbm.at[srcs[b]], rows[b], sems[b]).wait()
        rv = rows[b]
        wv = wrows[b]

        def edge_body(i, c2):
            wspl = wv[i, pl.ds(0, 16)]
            for j in range(D // 16):
                rv[i, pl.ds(j * 16, 16)] = rv[i, pl.ds(j * 16, 16)] * wspl
            return c2

        lax.fori_loop(0, CH, edge_body, 0, unroll=4)
        pltpu.sync_copy(rv, acc_sh.at[dsts[b]], add=True)

    fetch(0, 0)

    def pair_body(kk, carry):
        for b in range(2):
            cur = kk * 2 + b

            @pl.when(cur + 1 < CPT)
            def _():
                fetch(cur + 1, 1 - b)

            consume(b)
        return carry

    lax.fori_loop(0, CPT // 2, pair_body, 0)
    plsc.subcore_barrier()
    for bb in range(RPT // CH):
        pltpu.sync_copy(acc_sh.at[pl.ds(r0 + bb * CH, CH)], rows0)
        pltpu.sync_copy(rows0, out_hbm.at[cid, pl.ds(r0 + bb * CH, CH)])


_B = 2048
_WB = 16384


def _wrep_body(w_ref, out_ref):
    out_ref[...] = jnp.broadcast_to(w_ref[...], (_WB, 16))


def _wrep(w_p2):
    return pl.pallas_call(
        _wrep_body,
        grid=(EPAD // _WB,),
        in_specs=[pl.BlockSpec((_WB, 1), lambda i: (i, 0))],
        out_specs=pl.BlockSpec((_WB, 16), lambda i: (i, 0)),
        out_shape=jax.ShapeDtypeStruct((EPAD, 16), jnp.float32),
    )(w_p2)


def _tc1_body(gx_ref, tbl_ref, w1_ref, degp_ref, dinv_ref, hp_ref):
    gx = gx_ref[...]
    onehot = (gx == lax.broadcasted_iota(jnp.int32, (_B, NT), 1)).astype(jnp.float32)
    x0 = jnp.dot(onehot, tbl_ref[...], precision=HIGH)
    deg = degp_ref[0, :, 0:1] + degp_ref[1, :, 0:1] + 1.0
    dinv = lax.rsqrt(deg)
    dinv_ref[...] = dinv
    hp_ref[...] = jnp.dot(x0, w1_ref[...], precision=HIGH) * dinv


def _tc1(gx2, node_table, W1, degp):
    return pl.pallas_call(
        _tc1_body,
        grid=(NNP // _B,),
        in_specs=[
            pl.BlockSpec((_B, 1), lambda i: (i, 0)),
            pl.BlockSpec((NT, D), lambda i: (0, 0)),
            pl.BlockSpec((D, D), lambda i: (0, 0)),
            pl.BlockSpec((NC, _B, D), lambda i: (0, i, 0)),
        ],
        out_specs=[
            pl.BlockSpec((_B, 1), lambda i: (i, 0)),
            pl.BlockSpec((_B, D), lambda i: (i, 0)),
        ],
        out_shape=[
            jax.ShapeDtypeStruct((NNP, 1), jnp.float32),
            jax.ShapeDtypeStruct((NNP, D), jnp.float32),
        ],
    )(gx2, node_table, W1, degp)


def _tc_mid_body(acc_ref, hp_ref, dinv_ref, b_ref, wn_ref, out_ref):
    dinv = dinv_ref[...]
    x = dinv * (acc_ref[0] + acc_ref[1] + hp_ref[...]) + b_ref[...]
    x = jnp.maximum(x, 0.0)
    out_ref[...] = jnp.dot(x, wn_ref[...], precision=HIGH) * dinv


def _tc_mid(acc, hp, dinv, b, Wn):
    return pl.pallas_call(
        _tc_mid_body,
        grid=(NNP // _B,),
        in_specs=[
            pl.BlockSpec((NC, _B, D), lambda i: (0, i, 0)),
            pl.BlockSpec((_B, D), lambda i: (i, 0)),
            pl.BlockSpec((_B, 1), lambda i: (i, 0)),
            pl.BlockSpec((1, D), lambda i: (0, 0)),
            pl.BlockSpec((D, D), lambda i: (0, 0)),
        ],
        out_specs=pl.BlockSpec((_B, D), lambda i: (i, 0)),
        out_shape=jax.ShapeDtypeStruct((NNP, D), jnp.float32),
    )(acc, hp, dinv, b, Wn)


def _tc_fin_a_body(acc_ref, hp_ref, dinv_ref, b_ref, out_ref):
    dinv = dinv_ref[...]
    out_ref[...] = dinv * (acc_ref[0] + acc_ref[1] + hp_ref[...]) + b_ref[...]


def _tc_fin_a(acc, hp, dinv, b):
    return pl.pallas_call(
        _tc_fin_a_body,
        grid=(NNP // _B,),
        in_specs=[
            pl.BlockSpec((NC, _B, D), lambda i: (0, i, 0)),
            pl.BlockSpec((_B, D), lambda i: (i, 0)),
            pl.BlockSpec((_B, 1), lambda i: (i, 0)),
            pl.BlockSpec((1, D), lambda i: (0, 0)),
        ],
        out_specs=pl.BlockSpec((_B, D), lambda i: (i, 0)),
        out_shape=jax.ShapeDtypeStruct((NNP, D), jnp.float32),
    )(acc, hp, dinv, b)


_IR = NG * 3 * 7
_IC = 32 * 224
_IB = _IR // 8


def _img_body(img_ref, out_ref):
    r = lax.broadcasted_iota(jnp.int32, (_IC, 8), 0)
    c = lax.broadcasted_iota(jnp.int32, (_IC, 8), 1)
    q = jnp.where((r % 224) // 32 == c, 1.0 / 1024.0, 0.0)
    out_ref[...] = jnp.dot(img_ref[...], q, precision=HIGH)


def _img(img2d):
    return pl.pallas_call(
        _img_body,
        grid=(8,),
        in_specs=[pl.BlockSpec((_IB, _IC), lambda i: (i, 0))],
        out_specs=pl.BlockSpec((_IB, 8), lambda i: (i, 0)),
        out_shape=jax.ShapeDtypeStruct((_IR, 8), jnp.float32),
    )(img2d)


def _tc_fin_b_body(x3_ref, batch_ref, wg_ref, bg_ref, p147_ref, wimg_ref,
                   bimg_ref, wi_ref, bi_ref, oimg_ref, ogr_ref):
    onehot = (lax.broadcasted_iota(jnp.int32, (NG, NNP), 0)
              == batch_ref[...]).astype(jnp.float32)
    sums = jnp.dot(onehot, x3_ref[...], precision=HIGH)
    counts = jnp.sum(onehot, axis=1, keepdims=True)
    pooled = sums / jnp.maximum(counts, 1.0)
    xg = jnp.dot(pooled, wg_ref[...], precision=HIGH) + bg_ref[...]
    ogr_ref[...] = xg / jnp.sqrt(jnp.sum(xg * xg, axis=1, keepdims=True))
    wc = jnp.dot(wimg_ref[...], wi_ref[...], precision=HIGH)
    bc = jnp.dot(bimg_ref[...], wi_ref[...], precision=HIGH) + bi_ref[...]
    xi = jnp.dot(p147_ref[...], wc, precision=HIGH) + bc
    oimg_ref[...] = xi / jnp.sqrt(jnp.sum(xi * xi, axis=1, keepdims=True))


def _tc_fin_b(x3, batch2, Wg, bg, p147, Wimg, bimg, Wi, bi):
    return pl.pallas_call(
        _tc_fin_b_body,
        out_shape=[
            jax.ShapeDtypeStruct((NG, D), jnp.float32),
            jax.ShapeDtypeStruct((NG, D), jnp.float32),
        ],
    )(x3, batch2, Wg, bg, p147, Wimg, bimg, Wi, bi)


def kernel(images, graph_x, edge_index, edge_attr, batch, node_table,
           W1, b1, W2, b2, W3, b3, W_g, b_g, W_i, b_i, W_img, b_img):
    f32 = jnp.float32
    src = edge_index[0].astype(jnp.int32)
    dst = edge_index[1].astype(jnp.int32)
    w = edge_attr.astype(f32)
    npad = EPAD - NE
    src_p = jnp.concatenate([src, jnp.zeros((npad,), jnp.int32)])
    dst_p = jnp.concatenate([dst, jnp.zeros((npad,), jnp.int32)])
    w_p = jnp.concatenate([w, jnp.zeros((npad,), f32)])
    zeroD = jnp.zeros((NNP, D), f32)

    wrep = _wrep(w_p.reshape(EPAD, 1))
    degp = _sc_deg(dst_p, wrep, zeroD)
    gx2 = jnp.concatenate([graph_x.astype(jnp.int32),
                           jnp.zeros((NNP - NN,), jnp.int32)]).reshape(NNP, 1)
    dinv, hp1 = _tc1(gx2, node_table, W1, degp)

    acc1 = _sc_agg(hp1, src_p, dst_p, wrep, zeroD)
    hp2 = _tc_mid(acc1, hp1, dinv, b1.reshape(1, D), W2)
    acc2 = _sc_agg(hp2, src_p, dst_p, wrep, zeroD)
    hp3 = _tc_mid(acc2, hp2, dinv, b2.reshape(1, D), W3)
    acc3 = _sc_agg(hp3, src_p, dst_p, wrep, zeroD)
    x3 = _tc_fin_a(acc3, hp3, dinv, b3.reshape(1, D))

    img2d = images.reshape(_IR, _IC)
    pooledp = _img(img2d)
    p147 = pooledp[:, :7].reshape(NG, PF)

    out_images, out_graphs = _tc_fin_b(
        x3, jnp.concatenate([batch.astype(jnp.int32), jnp.full((NNP - NN,), NG, jnp.int32)]).reshape(1, NNP), W_g, b_g.reshape(1, D),
        p147, W_img, b_img.reshape(1, IMD), W_i, b_i.reshape(1, D))
    return (out_images, out_graphs)


# speedup vs baseline: 1.1937x; 1.1937x over previous
import functools

import jax
import jax.numpy as jnp
from jax import lax
from jax.experimental import pallas as pl
from jax.experimental.pallas import tpu as pltpu
from jax.experimental.pallas import tpu_sc as plsc

NG = 64
NN = 10000
NE = 320000
NT = 30
D = 128
IMD = 4096
PF = 147

NNP = 10240
NC = 2
NS = 16
NW = NC * NS
CH = 80
CPT = 128
EPAD = NW * CPT * CH
RPT = NNP // NS
HIGH = lax.Precision.HIGHEST

_mesh = plsc.VectorSubcoreMesh(core_axis_name="c", subcore_axis_name="s")


@functools.partial(
    pl.kernel,
    out_type=jax.ShapeDtypeStruct((NC, NNP, D), jnp.float32),
    mesh=_mesh,
    scratch_types=[
        pltpu.VMEM((CH,), jnp.int32),
        pltpu.VMEM((CH,), jnp.int32),
        pltpu.VMEM((CH, 16), jnp.float32),
        pltpu.VMEM((CH, 16), jnp.float32),
        pltpu.VMEM((CH, D), jnp.float32),
        pltpu.VMEM((CH, D), jnp.float32),
        pltpu.VMEM_SHARED((NNP, D), jnp.float32),
        pltpu.SemaphoreType.DMA,
        pltpu.SemaphoreType.DMA,
        pltpu.SemaphoreType.DMA,
        pltpu.SemaphoreType.DMA,
    ],
)
def _sc_deg(dst_hbm, wrep_hbm, zero_hbm, out_hbm,
            dst0, dst1, wrow0, wrow1, rows0, rows1,
            acc_sh, fsem0, fsem1, ssem0, ssem1):
    cid = lax.axis_index("c")
    sid = lax.axis_index("s")
    wid = sid * NC + cid
    r0 = sid * RPT
    dsts = (dst0, dst1)
    wrows = (wrow0, wrow1)
    rows = (rows0, rows1)
    fsems = (fsem0, fsem1)
    ssems = (ssem0, ssem1)

    pltpu.sync_copy(zero_hbm.at[pl.ds(0, CH)], rows0)
    for bb in range(RPT // CH):
        pltpu.sync_copy(rows0, acc_sh.at[pl.ds(r0 + bb * CH, CH)])
    plsc.subcore_barrier()

    def fetch(k, b):
        e0 = (wid * CPT + k) * CH
        pltpu.async_copy(dst_hbm.at[pl.ds(e0, CH)], dsts[b], fsems[b])
        pltpu.async_copy(wrep_hbm.at[pl.ds(e0, CH)], wrows[b], fsems[b])

    def consume(b):
        pltpu.make_async_copy(dst_hbm.at[pl.ds(0, CH)], dsts[b], fsems[b]).wait()
        pltpu.make_async_copy(wrep_hbm.at[pl.ds(0, CH)], wrows[b], fsems[b]).wait()
        rv = rows[b]
        wv = wrows[b]

        def edge_body(i, c2):
            wspl = wv[i, pl.ds(0, 16)]
            for j in range(D // 16):
                rv[i, pl.ds(j * 16, 16)] = wspl
            return c2

        lax.fori_loop(0, CH, edge_body, 0)
        pltpu.async_copy(rv, acc_sh.at[dsts[b]], ssems[b], add=True)

    fetch(0, 0)
    fetch(1, 1)

    def pair_body(kk, carry):
        for b in range(2):
            cur = kk * 2 + b
            consume(b)

            @pl.when(cur + 2 < CPT)
            def _():
                pltpu.make_async_copy(rows[b], acc_sh.at[dsts[b]], ssems[b]).wait()
                fetch(cur + 2, b)
        return carry

    lax.fori_loop(0, CPT // 2, pair_body, 0)
    pltpu.make_async_copy(rows0, acc_sh.at[dst0], ssem0).wait()
    pltpu.make_async_copy(rows1, acc_sh.at[dst1], ssem1).wait()
    plsc.subcore_barrier()
    for bb in range(RPT // CH):
        pltpu.sync_copy(acc_sh.at[pl.ds(r0 + bb * CH, CH)], rows0)
        pltpu.sync_copy(rows0, out_hbm.at[cid, pl.ds(r0 + bb * CH, CH)])


@functools.partial(
    pl.kernel,
    out_type=jax.ShapeDtypeStruct((NC, NNP, D), jnp.float32),
    mesh=_mesh,
    scratch_types=[
        pltpu.VMEM((CH,), jnp.int32),
        pltpu.VMEM((CH,), jnp.int32),
        pltpu.VMEM((CH,), jnp.int32),
        pltpu.VMEM((CH,), jnp.int32),
        pltpu.VMEM((CH, 16), jnp.float32),
        pltpu.VMEM((CH, 16), jnp.float32),
        pltpu.VMEM((CH, D), jnp.float32),
        pltpu.VMEM((CH, D), jnp.float32),
        pltpu.VMEM_SHARED((NNP, D), jnp.float32),
        pltpu.SemaphoreType.DMA,
        pltpu.SemaphoreType.DMA,
        pltpu.SemaphoreType.DMA,
        pltpu.SemaphoreType.DMA,
        pltpu.SemaphoreType.DMA,
        pltpu.SemaphoreType.DMA,
    ],
)
def _sc_agg(hp_hbm, src_hbm, dst_hbm, wrep_hbm, zero_hbm, out_hbm,
            src0, src1, dst0, dst1, wrow0, wrow1, rows0, rows1,
            acc_sh, fsem0, fsem1, gsem0, gsem1, ssem0, ssem1):
    cid = lax.axis_index("c")
    sid = lax.axis_index("s")
    wid = sid * NC + cid
    r0 = sid * RPT
    srcs = (src0, src1)
    dsts = (dst0, dst1)
    wrows = (wrow0, wrow1)
    rows = (rows0, rows1)
    fsems = (fsem0, fsem1)
    gsems = (gsem0, gsem1)
    ssems = (ssem0, ssem1)

    pltpu.sync_copy(zero_hbm.at[pl.ds(0, CH)], rows0)
    for bb in range(RPT // CH):
        pltpu.sync_copy(rows0, acc_sh.at[pl.ds(r0 + bb * CH, CH)])
    plsc.subcore_barrier()

    def fetch(k, b):
        e0 = (wid * CPT + k) * CH
        pltpu.sync_copy(src_hbm.at[pl.ds(e0, CH)], srcs[b])
        pltpu.async_copy(wrep_hbm.at[pl.ds(e0, CH)], wrows[b], fsems[b])
        pltpu.async_copy(dst_hbm.at[pl.ds(e0, CH)], dsts[b], fsems[b])
        pltpu.async_copy(hp_hbm.at[srcs[b]], rows[b], gsems[b])

    def consume(b):
        pltpu.make_async_copy(wrep_hbm.at[pl.ds(0, CH)], wrows[b], fsems[b]).wait()
        pltpu.make_async_copy(dst_hbm.at[pl.ds(0, CH)], dsts[b], fsems[b]).wait()
        pltpu.make_async_copy(hp_hbm.at[srcs[b]], rows[b], gsems[b]).wait()
        rv = rows[b]
        wv = wrows[b]

        def edge_body(i, c2):
            wspl = wv[i, pl.ds(0, 16)]
            for j in range(D // 16):
                rv[i, pl.ds(j * 16, 16)] = rv[i, pl.ds(j * 16, 16)] * wspl
            return c2

        lax.fori_loop(0, CH, edge_body, 0)
        pltpu.async_copy(rv, acc_sh.at[dsts[b]], ssems[b], add=True)

    fetch(0, 0)
    fetch(1, 1)

    def pair_body(kk, carry):
        for b in range(2):
            cur = kk * 2 + b
            consume(b)

            @pl.when(cur + 2 < CPT)
            def _():
                pltpu.make_async_copy(rows[b], acc_sh.at[dsts[b]], ssems[b]).wait()
                fetch(cur + 2, b)
        return carry

    lax.fori_loop(0, CPT // 2, pair_body, 0)
    pltpu.make_async_copy(rows0, acc_sh.at[dst0], ssem0).wait()
    pltpu.make_async_copy(rows1, acc_sh.at[dst1], ssem1).wait()
    plsc.subcore_barrier()
    for bb in range(RPT // CH):
        pltpu.sync_copy(acc_sh.at[pl.ds(r0 + bb * CH, CH)], rows0)
        pltpu.sync_copy(rows0, out_hbm.at[cid, pl.ds(r0 + bb * CH, CH)])


_B = 2048
_WB = 16384


def _wrep_body(w_ref, out_ref):
    out_ref[...] = jnp.broadcast_to(w_ref[...], (_WB, 16))


def _wrep(w_p2):
    return pl.pallas_call(
        _wrep_body,
        grid=(EPAD // _WB,),
        in_specs=[pl.BlockSpec((_WB, 1), lambda i: (i, 0))],
        out_specs=pl.BlockSpec((_WB, 16), lambda i: (i, 0)),
        out_shape=jax.ShapeDtypeStruct((EPAD, 16), jnp.float32),
    )(w_p2)


def _tc1_body(gx_ref, tbl_ref, w1_ref, degp_ref, dinv_ref, hp_ref):
    gx = gx_ref[...]
    onehot = (gx == lax.broadcasted_iota(jnp.int32, (_B, NT), 1)).astype(jnp.float32)
    x0 = jnp.dot(onehot, tbl_ref[...], precision=HIGH)
    deg = degp_ref[0, :, 0:1] + degp_ref[1, :, 0:1] + 1.0
    dinv = lax.rsqrt(deg)
    dinv_ref[...] = dinv
    hp_ref[...] = jnp.dot(x0, w1_ref[...], precision=HIGH) * dinv


def _tc1(gx2, node_table, W1, degp):
    return pl.pallas_call(
        _tc1_body,
        grid=(NNP // _B,),
        in_specs=[
            pl.BlockSpec((_B, 1), lambda i: (i, 0)),
            pl.BlockSpec((NT, D), lambda i: (0, 0)),
            pl.BlockSpec((D, D), lambda i: (0, 0)),
            pl.BlockSpec((NC, _B, D), lambda i: (0, i, 0)),
        ],
        out_specs=[
            pl.BlockSpec((_B, 1), lambda i: (i, 0)),
            pl.BlockSpec((_B, D), lambda i: (i, 0)),
        ],
        out_shape=[
            jax.ShapeDtypeStruct((NNP, 1), jnp.float32),
            jax.ShapeDtypeStruct((NNP, D), jnp.float32),
        ],
    )(gx2, node_table, W1, degp)


def _tc_mid_body(acc_ref, hp_ref, dinv_ref, b_ref, wn_ref, out_ref):
    dinv = dinv_ref[...]
    x = dinv * (acc_ref[0] + acc_ref[1] + hp_ref[...]) + b_ref[...]
    x = jnp.maximum(x, 0.0)
    out_ref[...] = jnp.dot(x, wn_ref[...], precision=HIGH) * dinv


def _tc_mid(acc, hp, dinv, b, Wn):
    return pl.pallas_call(
        _tc_mid_body,
        grid=(NNP // _B,),
        in_specs=[
            pl.BlockSpec((NC, _B, D), lambda i: (0, i, 0)),
            pl.BlockSpec((_B, D), lambda i: (i, 0)),
            pl.BlockSpec((_B, 1), lambda i: (i, 0)),
            pl.BlockSpec((1, D), lambda i: (0, 0)),
            pl.BlockSpec((D, D), lambda i: (0, 0)),
        ],
        out_specs=pl.BlockSpec((_B, D), lambda i: (i, 0)),
        out_shape=jax.ShapeDtypeStruct((NNP, D), jnp.float32),
    )(acc, hp, dinv, b, Wn)


def _tc_fin_a_body(acc_ref, hp_ref, dinv_ref, b_ref, out_ref):
    dinv = dinv_ref[...]
    out_ref[...] = dinv * (acc_ref[0] + acc_ref[1] + hp_ref[...]) + b_ref[...]


def _tc_fin_a(acc, hp, dinv, b):
    return pl.pallas_call(
        _tc_fin_a_body,
        grid=(NNP // _B,),
        in_specs=[
            pl.BlockSpec((NC, _B, D), lambda i: (0, i, 0)),
            pl.BlockSpec((_B, D), lambda i: (i, 0)),
            pl.BlockSpec((_B, 1), lambda i: (i, 0)),
            pl.BlockSpec((1, D), lambda i: (0, 0)),
        ],
        out_specs=pl.BlockSpec((_B, D), lambda i: (i, 0)),
        out_shape=jax.ShapeDtypeStruct((NNP, D), jnp.float32),
    )(acc, hp, dinv, b)


_IR = NG * 3 * 7
_IC = 32 * 224
_IB = _IR // 8


def _img_body(img_ref, out_ref):
    r = lax.broadcasted_iota(jnp.int32, (_IC, 8), 0)
    c = lax.broadcasted_iota(jnp.int32, (_IC, 8), 1)
    q = jnp.where((r % 224) // 32 == c, 1.0 / 1024.0, 0.0)
    out_ref[...] = jnp.dot(img_ref[...], q, precision=HIGH)


def _img(img2d):
    return pl.pallas_call(
        _img_body,
        grid=(8,),
        in_specs=[pl.BlockSpec((_IB, _IC), lambda i: (i, 0))],
        out_specs=pl.BlockSpec((_IB, 8), lambda i: (i, 0)),
        out_shape=jax.ShapeDtypeStruct((_IR, 8), jnp.float32),
    )(img2d)


def _tc_fin_b_body(x3_ref, batch_ref, wg_ref, bg_ref, p147_ref, wimg_ref,
                   bimg_ref, wi_ref, bi_ref, oimg_ref, ogr_ref):
    onehot = (lax.broadcasted_iota(jnp.int32, (NG, NNP), 0)
              == batch_ref[...]).astype(jnp.float32)
    sums = jnp.dot(onehot, x3_ref[...], precision=HIGH)
    counts = jnp.sum(onehot, axis=1, keepdims=True)
    pooled = sums / jnp.maximum(counts, 1.0)
    xg = jnp.dot(pooled, wg_ref[...], precision=HIGH) + bg_ref[...]
    ogr_ref[...] = xg / jnp.sqrt(jnp.sum(xg * xg, axis=1, keepdims=True))
    wc = jnp.dot(wimg_ref[...], wi_ref[...], precision=HIGH)
    bc = jnp.dot(bimg_ref[...], wi_ref[...], precision=HIGH) + bi_ref[...]
    xi = jnp.dot(p147_ref[...], wc, precision=HIGH) + bc
    oimg_ref[...] = xi / jnp.sqrt(jnp.sum(xi * xi, axis=1, keepdims=True))


def _tc_fin_b(x3, batch2, Wg, bg, p147, Wimg, bimg, Wi, bi):
    return pl.pallas_call(
        _tc_fin_b_body,
        out_shape=[
            jax.ShapeDtypeStruct((NG, D), jnp.float32),
            jax.ShapeDtypeStruct((NG, D), jnp.float32),
        ],
    )(x3, batch2, Wg, bg, p147, Wimg, bimg, Wi, bi)


def kernel(images, graph_x, edge_index, edge_attr, batch, node_table,
           W1, b1, W2, b2, W3, b3, W_g, b_g, W_i, b_i, W_img, b_img):
    f32 = jnp.float32
    src = edge_index[0].astype(jnp.int32)
    dst = edge_index[1].astype(jnp.int32)
    w = edge_attr.astype(f32)
    npad = EPAD - NE
    src_p = jnp.concatenate([src, jnp.zeros((npad,), jnp.int32)])
    dst_p = jnp.concatenate([dst, jnp.zeros((npad,), jnp.int32)])
    w_p = jnp.concatenate([w, jnp.zeros((npad,), f32)])
    zeroD = jnp.zeros((NNP, D), f32)

    wrep = _wrep(w_p.reshape(EPAD, 1))
    degp = _sc_deg(dst_p, wrep, zeroD)
    gx2 = jnp.concatenate([graph_x.astype(jnp.int32),
                           jnp.zeros((NNP - NN,), jnp.int32)]).reshape(NNP, 1)
    dinv, hp1 = _tc1(gx2, node_table, W1, degp)

    acc1 = _sc_agg(hp1, src_p, dst_p, wrep, zeroD)
    hp2 = _tc_mid(acc1, hp1, dinv, b1.reshape(1, D), W2)
    acc2 = _sc_agg(hp2, src_p, dst_p, wrep, zeroD)
    hp3 = _tc_mid(acc2, hp2, dinv, b2.reshape(1, D), W3)
    acc3 = _sc_agg(hp3, src_p, dst_p, wrep, zeroD)
    x3 = _tc_fin_a(acc3, hp3, dinv, b3.reshape(1, D))

    img2d = images.reshape(_IR, _IC)
    pooledp = _img(img2d)
    p147 = pooledp[:, :7].reshape(NG, PF)

    out_images, out_graphs = _tc_fin_b(
        x3, jnp.concatenate([batch.astype(jnp.int32), jnp.full((NNP - NN,), NG, jnp.int32)]).reshape(1, NNP), W_g, b_g.reshape(1, D),
        p147, W_img, b_img.reshape(1, IMD), W_i, b_i.reshape(1, D))
    return (out_images, out_graphs)
